# Initial kernel scaffold; baseline (speedup 1.0000x reference)
import jax
import jax.numpy as jnp
from jax.experimental import pallas as pl


def kernel(feat1, feat2, edge_index1, edge_index2, W0, b0, W1, b1):
    raise NotImplementedError("write your pallas kernel here")


# speedup vs baseline: 2.9315x; 2.9315x over previous
import functools

import jax
import jax.numpy as jnp
from jax import lax
from jax.experimental import pallas as pl
from jax.experimental.pallas import tpu as pltpu
from jax.experimental.pallas import tpu_sc as plsc

N = 10000
D = 128
NE = 320000
NP = 10240
DUMMY = NP - 1
NCORES = 2
NSUB = 16
TILES = NCORES * NSUB
CH = 128
CPT = 158
NE_PAD = NSUB * CPT * CH
RPT = NP // NSUB
ZROWS = 64

_MESH = plsc.VectorSubcoreMesh(
    core_axis_name="c", subcore_axis_name="s", num_cores=NCORES,
    num_subcores=NSUB)


@functools.partial(
    pl.kernel,
    out_type=jax.ShapeDtypeStruct((2 * NP, D), jnp.float32),
    mesh=_MESH,
    scratch_types=[
        pltpu.VMEM((CH, D), jnp.float32),
        pltpu.VMEM((CH, D), jnp.float32),
        pltpu.VMEM((ZROWS, D), jnp.float32),
        pltpu.VMEM((CH,), jnp.int32),
        pltpu.VMEM((CH,), jnp.int32),
        pltpu.VMEM_SHARED((NP, D), jnp.float32),
    ],
)
def _deg_kernel(edges_hbm, onesl_hbm, onesr_hbm, zeros_hbm, out_hbm, onesl_v,
                onesr_v, zbuf_v, sidx_v, didx_v, acc_s):
    c = lax.axis_index("c")
    s = lax.axis_index("s")
    pltpu.sync_copy(onesl_hbm, onesl_v)
    pltpu.sync_copy(onesr_hbm, onesr_v)
    pltpu.sync_copy(zeros_hbm, zbuf_v)

    base = s * RPT
    for k in range(RPT // ZROWS):
        pltpu.sync_copy(zbuf_v, acc_s.at[pl.ds(base + k * ZROWS, ZROWS)])
    plsc.subcore_barrier()

    def body(j, _):
        row = (c * 2 * NSUB + s) * CPT + j
        pltpu.sync_copy(edges_hbm.at[row], sidx_v)
        row = ((c * 2 + 1) * NSUB + s) * CPT + j
        pltpu.sync_copy(edges_hbm.at[row], didx_v)
        pltpu.sync_copy(onesl_v, acc_s.at[sidx_v], add=True)
        pltpu.sync_copy(onesr_v, acc_s.at[didx_v], add=True)
        return 0

    lax.fori_loop(0, CPT, body, 0)
    plsc.subcore_barrier()

    for k in range(RPT // ZROWS):
        rows = pl.ds(base + k * ZROWS, ZROWS)
        out = pl.ds(c * NP + base + k * ZROWS, ZROWS)
        pltpu.sync_copy(acc_s.at[rows], zbuf_v)
        pltpu.sync_copy(zbuf_v, out_hbm.at[out])


@functools.partial(
    pl.kernel,
    out_type=jax.ShapeDtypeStruct((2 * NP, D), jnp.float32),
    mesh=_MESH,
    scratch_types=[
        pltpu.VMEM((CH,), jnp.int32),
        pltpu.VMEM((CH,), jnp.int32),
        pltpu.VMEM((CH, D), jnp.float32),
        pltpu.VMEM((ZROWS, D), jnp.float32),
        pltpu.VMEM_SHARED((NP, D), jnp.float32),
        pltpu.SemaphoreType.DMA,
    ],
)
def _edge_kernel(y_hbm, src_hbm, dst_hbm, zeros_hbm, out_hbm, sidx_v, didx_v,
                 rows_v, zbuf_v, acc_s, sem):
    c = lax.axis_index("c")
    s = lax.axis_index("s")
    pltpu.sync_copy(zeros_hbm, zbuf_v)

    base = s * RPT
    for k in range(RPT // ZROWS):
        pltpu.sync_copy(zbuf_v, acc_s.at[pl.ds(base + k * ZROWS, ZROWS)])
    plsc.subcore_barrier()

    def body(j, _):
        row = (c * NSUB + s) * CPT + j
        pltpu.sync_copy(src_hbm.at[row], sidx_v)
        pltpu.sync_copy(dst_hbm.at[row], didx_v)
        pltpu.async_copy(y_hbm.at[sidx_v], rows_v, sem).wait()
        pltpu.sync_copy(rows_v, acc_s.at[didx_v], add=True)
        return 0

    lax.fori_loop(0, CPT, body, 0)
    plsc.subcore_barrier()

    for k in range(RPT // ZROWS):
        rows = pl.ds(base + k * ZROWS, ZROWS)
        out = pl.ds(c * NP + base + k * ZROWS, ZROWS)
        pltpu.sync_copy(acc_s.at[rows], zbuf_v)
        pltpu.sync_copy(zbuf_v, out_hbm.at[out])


def _scales(d_ref):
    dout = d_ref[0, 0:N, 0:1]
    din = d_ref[0, 0:N, 64:65]
    s_out = lax.rsqrt(jnp.maximum(dout, 1.0))
    s_in = lax.rsqrt(jnp.maximum(din, 1.0))
    return s_out, s_in


def _tc_pre_body(x_ref, d_ref, o_ref):
    s_out, _ = _scales(d_ref)
    o_ref[0, 0:N, :] = x_ref[0] * s_out


_tc_pre = pl.pallas_call(
    _tc_pre_body,
    grid=(2,),
    in_specs=[
        pl.BlockSpec((1, N, D), lambda g: (g, 0, 0)),
        pl.BlockSpec((1, NP, D), lambda g: (g, 0, 0)),
    ],
    out_specs=pl.BlockSpec((1, NP, D), lambda g: (g, 0, 0)),
    out_shape=jax.ShapeDtypeStruct((2, NP, D), jnp.float32),
)


def _tc_mid_body(a_ref, d_ref, b_ref, w_ref, o_ref):
    s_out, s_in = _scales(d_ref)
    y = jnp.dot(a_ref[0, 0:N, :] * s_in, w_ref[...],
                preferred_element_type=jnp.float32)
    h = jnp.maximum(y + b_ref[0:1, :], 0.0)
    o_ref[0, 0:N, :] = h * s_out


_tc_mid = pl.pallas_call(
    _tc_mid_body,
    grid=(2,),
    in_specs=[
        pl.BlockSpec((1, NP, D), lambda g: (g, 0, 0)),
        pl.BlockSpec((1, NP, D), lambda g: (g, 0, 0)),
        pl.BlockSpec((8, D), lambda g: (0, 0)),
        pl.BlockSpec((D, D), lambda g: (0, 0)),
    ],
    out_specs=pl.BlockSpec((1, NP, D), lambda g: (g, 0, 0)),
    out_shape=jax.ShapeDtypeStruct((2, NP, D), jnp.float32),
)


def _tc_post_body(a_ref, d_ref, b_ref, w_ref, o_ref):
    _, s_in = _scales(d_ref)
    f = jnp.dot(a_ref[0, 0:N, :] * s_in, w_ref[...],
                preferred_element_type=jnp.float32) + b_ref[0:1, :]
    mu = jnp.mean(f, axis=0, keepdims=True)
    d = f - mu
    var = jnp.sum(d * d, axis=0, keepdims=True) * (1.0 / (N - 1))
    o_ref[0] = d / jnp.sqrt(var)


_tc_post = pl.pallas_call(
    _tc_post_body,
    grid=(2,),
    in_specs=[
        pl.BlockSpec((1, NP, D), lambda g: (g, 0, 0)),
        pl.BlockSpec((1, NP, D), lambda g: (g, 0, 0)),
        pl.BlockSpec((8, D), lambda g: (0, 0)),
        pl.BlockSpec((D, D), lambda g: (0, 0)),
    ],
    out_specs=pl.BlockSpec((1, N, D), lambda g: (g, 0, 0)),
    out_shape=jax.ShapeDtypeStruct((2, N, D), jnp.float32),
)


def kernel(feat1, feat2, edge_index1, edge_index2, W0, b0, W1, b1):
    npad = NE_PAD - NE

    def padv(v, val):
        return jnp.concatenate([v, jnp.full((npad,), val, jnp.int32)])

    src1, dst1 = edge_index1[0], edge_index1[1]
    src2, dst2 = edge_index2[0], edge_index2[1]

    e_deg = jnp.stack([
        jnp.stack([padv(src1, DUMMY), padv(dst1, DUMMY)]),
        jnp.stack([padv(src2, DUMMY), padv(dst2, DUMMY)]),
    ]).reshape(2 * 2 * NSUB * CPT, CH)
    srcg = jnp.stack([padv(src1, 0), padv(src2, 0) + NP]) \
        .reshape(2 * NSUB * CPT, CH)
    dstg = jnp.stack([padv(dst1, DUMMY), padv(dst2, DUMMY)]) \
        .reshape(2 * NSUB * CPT, CH)

    col = jnp.arange(D, dtype=jnp.int32)
    onesl = jnp.broadcast_to((col < 64).astype(jnp.float32), (CH, D))
    onesr = jnp.broadcast_to((col >= 64).astype(jnp.float32), (CH, D))
    zerosd = jnp.zeros((ZROWS, D), jnp.float32)

    degs = _deg_kernel(e_deg, onesl, onesr, zerosd).reshape(2, NP, D)
    b0t = jnp.tile(b0[None, :], (8, 1))
    b1t = jnp.tile(b1[None, :], (8, 1))

    xs = jnp.stack([feat1, feat2])
    ys = _tc_pre(xs, degs)
    agg1 = _edge_kernel(ys.reshape(2 * NP, D), srcg, dstg,
                        zerosd).reshape(2, NP, D)
    ys2 = _tc_mid(agg1, degs, b0t, W0)
    agg2 = _edge_kernel(ys2.reshape(2 * NP, D), srcg, dstg,
                        zerosd).reshape(2, NP, D)
    z = _tc_post(agg2, degs, b1t, W1)
    return z[0], z[1]


# speedup vs baseline: 4.6736x; 1.5943x over previous
import functools

import jax
import jax.numpy as jnp
from jax import lax
from jax.experimental import pallas as pl
from jax.experimental.pallas import tpu as pltpu
from jax.experimental.pallas import tpu_sc as plsc

N = 10000
D = 128
NE = 320000
NP = 10240
DUMMY = NP - 1
NCORES = 2
NSUB = 16
CH = 128
CPT = 158
NE_PAD = NSUB * CPT * CH
RPT = NP // NSUB
ZROWS = 32

_MESH = plsc.VectorSubcoreMesh(
    core_axis_name="c", subcore_axis_name="s", num_cores=NCORES,
    num_subcores=NSUB)


def _zero_acc(zbuf_v, acc_s, base):
    for k in range(RPT // ZROWS):
        pltpu.sync_copy(zbuf_v, acc_s.at[pl.ds(base + k * ZROWS, ZROWS)])


def _copy_out(acc_s, zbuf_v, out_hbm, base, obase):
    for k in range(RPT // ZROWS):
        rows = pl.ds(base + k * ZROWS, ZROWS)
        out = pl.ds(obase + k * ZROWS, ZROWS)
        pltpu.sync_copy(acc_s.at[rows], zbuf_v)
        pltpu.sync_copy(zbuf_v, out_hbm.at[out])


@functools.partial(
    pl.kernel,
    out_type=jax.ShapeDtypeStruct((2 * NP, D), jnp.float32),
    mesh=_MESH,
    scratch_types=[
        pltpu.VMEM((CH, D), jnp.float32),
        pltpu.VMEM((CH, D), jnp.float32),
        pltpu.VMEM((ZROWS, D), jnp.float32),
        pltpu.VMEM((2, 2, CH), jnp.int32),
        pltpu.VMEM_SHARED((NP, D), jnp.float32),
        pltpu.SemaphoreType.DMA,
        pltpu.SemaphoreType.DMA,
    ],
)
def _deg_kernel(edges_hbm, onesl_hbm, onesr_hbm, zeros_hbm, out_hbm, onesl_v,
                onesr_v, zbuf_v, islot_v, acc_s, isem0, isem1):
    c = lax.axis_index("c")
    s = lax.axis_index("s")
    pltpu.sync_copy(onesl_hbm, onesl_v)
    pltpu.sync_copy(onesr_hbm, onesr_v)
    pltpu.sync_copy(zeros_hbm, zbuf_v)
    ibase = (c * NSUB + s) * 2 * CPT
    pltpu.sync_copy(edges_hbm.at[ibase], islot_v.at[0, 0])
    pltpu.sync_copy(edges_hbm.at[ibase + 1], islot_v.at[0, 1])
    pltpu.async_copy(edges_hbm.at[ibase + 2], islot_v.at[1, 0], isem1)
    pltpu.async_copy(edges_hbm.at[ibase + 3], islot_v.at[1, 1], isem1)

    base = s * RPT
    _zero_acc(zbuf_v, acc_s, base)
    plsc.subcore_barrier()

    isems = (isem0, isem1)

    def body(k, _):
        for p in range(2):
            j = 2 * k + p

            @pl.when(j >= 1)
            def _():
                pltpu.make_async_copy(edges_hbm.at[0], islot_v.at[p, 0],
                                      isems[p]).wait()
                pltpu.make_async_copy(edges_hbm.at[0], islot_v.at[p, 1],
                                      isems[p]).wait()

            pltpu.sync_copy(onesl_v, acc_s.at[islot_v.at[p, 0]], add=True)
            pltpu.sync_copy(onesr_v, acc_s.at[islot_v.at[p, 1]], add=True)

            @pl.when(j + 2 < CPT)
            def _():
                r = ibase + 2 * (j + 2)
                pltpu.async_copy(edges_hbm.at[r], islot_v.at[p, 0], isems[p])
                pltpu.async_copy(edges_hbm.at[r + 1], islot_v.at[p, 1],
                                 isems[p])
        return 0

    lax.fori_loop(0, CPT // 2, body, 0)
    plsc.subcore_barrier()
    _copy_out(acc_s, zbuf_v, out_hbm, base, c * NP + base)


@functools.partial(
    pl.kernel,
    out_type=jax.ShapeDtypeStruct((2 * NP, D), jnp.float32),
    mesh=_MESH,
    scratch_types=[
        pltpu.VMEM((2, 2, CH), jnp.int32),
        pltpu.VMEM((CH, D), jnp.float32),
        pltpu.VMEM((CH, D), jnp.float32),
        pltpu.VMEM((ZROWS, D), jnp.float32),
        pltpu.VMEM_SHARED((NP, D), jnp.float32),
        pltpu.SemaphoreType.DMA,
        pltpu.SemaphoreType.DMA,
        pltpu.SemaphoreType.DMA,
        pltpu.SemaphoreType.DMA,
    ],
)
def _edge_kernel(y_hbm, ei_hbm, zeros_hbm, out_hbm, islot_v, rows0_v, rows1_v,
                 zbuf_v, acc_s, gsem0, gsem1, isem0, isem1):
    c = lax.axis_index("c")
    s = lax.axis_index("s")
    pltpu.sync_copy(zeros_hbm, zbuf_v)
    ibase = (c * NSUB + s) * 2 * CPT
    pltpu.sync_copy(ei_hbm.at[ibase], islot_v.at[0, 0])
    pltpu.sync_copy(ei_hbm.at[ibase + 1], islot_v.at[0, 1])
    pltpu.async_copy(y_hbm.at[islot_v.at[0, 0]], rows0_v, gsem0)
    pltpu.async_copy(ei_hbm.at[ibase + 2], islot_v.at[1, 0], isem1)
    pltpu.async_copy(ei_hbm.at[ibase + 3], islot_v.at[1, 1], isem1)

    base = s * RPT
    _zero_acc(zbuf_v, acc_s, base)
    plsc.subcore_barrier()

    bufs = ((rows0_v, gsem0, isem0), (rows1_v, gsem1, isem1))

    def body(k, _):
        for p in range(2):
            j = 2 * k + p
            rows_cur, gsem_cur, isem_cur = bufs[p]
            rows_nxt, gsem_nxt, isem_nxt = bufs[1 - p]

            @pl.when(j + 1 < CPT)
            def _():
                pltpu.make_async_copy(ei_hbm.at[0], islot_v.at[1 - p, 0],
                                      isem_nxt).wait()
                pltpu.make_async_copy(ei_hbm.at[0], islot_v.at[1 - p, 1],
                                      isem_nxt).wait()
                pltpu.async_copy(y_hbm.at[islot_v.at[1 - p, 0]], rows_nxt,
                                 gsem_nxt)

            pltpu.make_async_copy(y_hbm.at[islot_v.at[p, 0]], rows_cur,
                                  gsem_cur).wait()
            pltpu.sync_copy(rows_cur, acc_s.at[islot_v.at[p, 1]], add=True)

            @pl.when(j + 2 < CPT)
            def _():
                r = ibase + 2 * (j + 2)
                pltpu.async_copy(ei_hbm.at[r], islot_v.at[p, 0], isem_cur)
                pltpu.async_copy(ei_hbm.at[r + 1], islot_v.at[p, 1], isem_cur)
        return 0

    lax.fori_loop(0, CPT // 2, body, 0)
    plsc.subcore_barrier()
    _copy_out(acc_s, zbuf_v, out_hbm, base, c * NP + base)


def _scales(d_ref):
    dout = d_ref[0, 0:N, 0:1]
    din = d_ref[0, 0:N, 64:65]
    s_out = lax.rsqrt(jnp.maximum(dout, 1.0))
    s_in = lax.rsqrt(jnp.maximum(din, 1.0))
    return s_out, s_in


def _tc_pre_body(x_ref, d_ref, o_ref):
    s_out, _ = _scales(d_ref)
    o_ref[0, 0:N, :] = x_ref[0] * s_out


_tc_pre = pl.pallas_call(
    _tc_pre_body,
    grid=(2,),
    in_specs=[
        pl.BlockSpec((1, N, D), lambda g: (g, 0, 0)),
        pl.BlockSpec((1, NP, D), lambda g: (g, 0, 0)),
    ],
    out_specs=pl.BlockSpec((1, NP, D), lambda g: (g, 0, 0)),
    out_shape=jax.ShapeDtypeStruct((2, NP, D), jnp.float32),
)


def _tc_mid_body(a_ref, d_ref, b_ref, w_ref, o_ref):
    s_out, s_in = _scales(d_ref)
    y = jnp.dot(a_ref[0, 0:N, :] * s_in, w_ref[...],
                preferred_element_type=jnp.float32)
    h = jnp.maximum(y + b_ref[0:1, :], 0.0)
    o_ref[0, 0:N, :] = h * s_out


_tc_mid = pl.pallas_call(
    _tc_mid_body,
    grid=(2,),
    in_specs=[
        pl.BlockSpec((1, NP, D), lambda g: (g, 0, 0)),
        pl.BlockSpec((1, NP, D), lambda g: (g, 0, 0)),
        pl.BlockSpec((8, D), lambda g: (0, 0)),
        pl.BlockSpec((D, D), lambda g: (0, 0)),
    ],
    out_specs=pl.BlockSpec((1, NP, D), lambda g: (g, 0, 0)),
    out_shape=jax.ShapeDtypeStruct((2, NP, D), jnp.float32),
)


def _tc_post_body(a_ref, d_ref, b_ref, w_ref, o_ref):
    _, s_in = _scales(d_ref)
    f = jnp.dot(a_ref[0, 0:N, :] * s_in, w_ref[...],
                preferred_element_type=jnp.float32) + b_ref[0:1, :]
    mu = jnp.mean(f, axis=0, keepdims=True)
    d = f - mu
    var = jnp.sum(d * d, axis=0, keepdims=True) * (1.0 / (N - 1))
    o_ref[0] = d / jnp.sqrt(var)


_tc_post = pl.pallas_call(
    _tc_post_body,
    grid=(2,),
    in_specs=[
        pl.BlockSpec((1, NP, D), lambda g: (g, 0, 0)),
        pl.BlockSpec((1, NP, D), lambda g: (g, 0, 0)),
        pl.BlockSpec((8, D), lambda g: (0, 0)),
        pl.BlockSpec((D, D), lambda g: (0, 0)),
    ],
    out_specs=pl.BlockSpec((1, N, D), lambda g: (g, 0, 0)),
    out_shape=jax.ShapeDtypeStruct((2, N, D), jnp.float32),
)


def kernel(feat1, feat2, edge_index1, edge_index2, W0, b0, W1, b1):
    npad = NE_PAD - NE

    def padv(v, val):
        return jnp.concatenate([v, jnp.full((npad,), val, jnp.int32)])

    src1, dst1 = edge_index1[0], edge_index1[1]
    src2, dst2 = edge_index2[0], edge_index2[1]

    def pack(sa, da):
        return jnp.stack([sa.reshape(NSUB * CPT, CH),
                          da.reshape(NSUB * CPT, CH)],
                         axis=1).reshape(2 * NSUB * CPT, CH)

    e_deg = jnp.concatenate([
        pack(padv(src1, DUMMY), padv(dst1, DUMMY)),
        pack(padv(src2, DUMMY), padv(dst2, DUMMY)),
    ])
    ei_gs = jnp.concatenate([
        pack(padv(src1, 0), padv(dst1, DUMMY)),
        pack(padv(src2, 0) + NP, padv(dst2, DUMMY)),
    ])

    col = jnp.arange(D, dtype=jnp.int32)
    onesl = jnp.broadcast_to((col < 64).astype(jnp.float32), (CH, D))
    onesr = jnp.broadcast_to((col >= 64).astype(jnp.float32), (CH, D))
    zerosd = jnp.zeros((ZROWS, D), jnp.float32)

    degs = _deg_kernel(e_deg, onesl, onesr, zerosd).reshape(2, NP, D)
    b0t = jnp.tile(b0[None, :], (8, 1))
    b1t = jnp.tile(b1[None, :], (8, 1))

    xs = jnp.stack([feat1, feat2])
    ys = _tc_pre(xs, degs)
    agg1 = _edge_kernel(ys.reshape(2 * NP, D), ei_gs,
                        zerosd).reshape(2, NP, D)
    ys2 = _tc_mid(agg1, degs, b0t, W0)
    agg2 = _edge_kernel(ys2.reshape(2 * NP, D), ei_gs,
                        zerosd).reshape(2, NP, D)
    z = _tc_post(agg2, degs, b1t, W1)
    return z[0], z[1]


# speedup vs baseline: 4.8814x; 1.0445x over previous
import functools

import jax
import jax.numpy as jnp
from jax import lax
from jax.experimental import pallas as pl
from jax.experimental.pallas import tpu as pltpu
from jax.experimental.pallas import tpu_sc as plsc

N = 10000
D = 128
NE = 320000
NP = 10240
DUMMY = NP - 1
NCORES = 2
NSUB = 16
CH = 128
CPT = 158
NE_PAD = NSUB * CPT * CH
RPT = NP // NSUB
ZROWS = 32

_MESH = plsc.VectorSubcoreMesh(
    core_axis_name="c", subcore_axis_name="s", num_cores=NCORES,
    num_subcores=NSUB)


def _zero_acc(zbuf_v, acc_s, base):
    for k in range(RPT // ZROWS):
        pltpu.sync_copy(zbuf_v, acc_s.at[pl.ds(base + k * ZROWS, ZROWS)])


def _copy_out(acc_s, zbuf_v, out_hbm, base, obase):
    for k in range(RPT // ZROWS):
        rows = pl.ds(base + k * ZROWS, ZROWS)
        out = pl.ds(obase + k * ZROWS, ZROWS)
        pltpu.sync_copy(acc_s.at[rows], zbuf_v)
        pltpu.sync_copy(zbuf_v, out_hbm.at[out])


@functools.partial(
    pl.kernel,
    out_type=jax.ShapeDtypeStruct((2 * NP, D), jnp.float32),
    mesh=_MESH,
    scratch_types=[
        pltpu.VMEM((CH, D), jnp.float32),
        pltpu.VMEM((CH, D), jnp.float32),
        pltpu.VMEM((ZROWS, D), jnp.float32),
        pltpu.VMEM((2, 2, CH), jnp.int32),
        pltpu.VMEM_SHARED((NP, D), jnp.float32),
        pltpu.SemaphoreType.DMA,
        pltpu.SemaphoreType.DMA,
    ],
)
def _deg_kernel(edges_hbm, onesl_hbm, onesr_hbm, zeros_hbm, out_hbm, onesl_v,
                onesr_v, zbuf_v, islot_v, acc_s, isem0, isem1):
    c = lax.axis_index("c")
    s = lax.axis_index("s")
    pltpu.sync_copy(onesl_hbm, onesl_v)
    pltpu.sync_copy(onesr_hbm, onesr_v)
    pltpu.sync_copy(zeros_hbm, zbuf_v)
    ibase = (c * NSUB + s) * 2 * CPT
    pltpu.sync_copy(edges_hbm.at[ibase], islot_v.at[0, 0])
    pltpu.sync_copy(edges_hbm.at[ibase + 1], islot_v.at[0, 1])
    pltpu.async_copy(edges_hbm.at[ibase + 2], islot_v.at[1, 0], isem1)
    pltpu.async_copy(edges_hbm.at[ibase + 3], islot_v.at[1, 1], isem1)

    base = s * RPT
    _zero_acc(zbuf_v, acc_s, base)
    plsc.subcore_barrier()

    isems = (isem0, isem1)

    def body(k, _):
        for p in range(2):
            j = 2 * k + p

            @pl.when(j >= 1)
            def _():
                pltpu.make_async_copy(edges_hbm.at[0], islot_v.at[p, 0],
                                      isems[p]).wait()
                pltpu.make_async_copy(edges_hbm.at[0], islot_v.at[p, 1],
                                      isems[p]).wait()

            pltpu.sync_copy(onesl_v, acc_s.at[islot_v.at[p, 0]], add=True)
            pltpu.sync_copy(onesr_v, acc_s.at[islot_v.at[p, 1]], add=True)

            @pl.when(j + 2 < CPT)
            def _():
                r = ibase + 2 * (j + 2)
                pltpu.async_copy(edges_hbm.at[r], islot_v.at[p, 0], isems[p])
                pltpu.async_copy(edges_hbm.at[r + 1], islot_v.at[p, 1],
                                 isems[p])
        return 0

    lax.fori_loop(0, CPT // 2, body, 0)
    plsc.subcore_barrier()
    _copy_out(acc_s, zbuf_v, out_hbm, base, c * NP + base)


@functools.partial(
    pl.kernel,
    out_type=jax.ShapeDtypeStruct((2 * NP, D), jnp.float32),
    mesh=_MESH,
    scratch_types=[
        pltpu.VMEM((4, 2, CH), jnp.int32),
        pltpu.VMEM((CH, D), jnp.float32),
        pltpu.VMEM((CH, D), jnp.float32),
        pltpu.VMEM((ZROWS, D), jnp.float32),
        pltpu.VMEM_SHARED((NP, D), jnp.float32),
        pltpu.SemaphoreType.DMA,
        pltpu.SemaphoreType.DMA,
        pltpu.SemaphoreType.DMA,
        pltpu.SemaphoreType.DMA,
        pltpu.SemaphoreType.DMA,
        pltpu.SemaphoreType.DMA,
    ],
)
def _edge_kernel(y_hbm, ei_hbm, zeros_hbm, out_hbm, islot_v, rows0_v, rows1_v,
                 zbuf_v, acc_s, gsem0, gsem1, isem0, isem1, ssem0, ssem1):
    c = lax.axis_index("c")
    s = lax.axis_index("s")
    pltpu.sync_copy(zeros_hbm, zbuf_v)
    ibase = (c * NSUB + s) * 2 * CPT
    pltpu.sync_copy(ei_hbm.at[ibase], islot_v.at[0, 0])
    pltpu.sync_copy(ei_hbm.at[ibase + 1], islot_v.at[0, 1])
    pltpu.async_copy(y_hbm.at[islot_v.at[0, 0]], rows0_v, gsem0)
    pltpu.async_copy(ei_hbm.at[ibase + 2], islot_v.at[1, 0], isem1)
    pltpu.async_copy(ei_hbm.at[ibase + 3], islot_v.at[1, 1], isem1)

    base = s * RPT
    _zero_acc(zbuf_v, acc_s, base)
    plsc.subcore_barrier()

    rows = (rows0_v, rows1_v)
    gsems = (gsem0, gsem1)
    isems = (isem0, isem1)
    ssems = (ssem0, ssem1)

    def cond(flag, fn):
        if isinstance(flag, bool):
            if flag:
                fn()
        else:
            pl.when(flag)(fn)

    def phase(j, jp):
        p = jp % 2
        sl_cur, sl_nxt, sl_pre = jp, (jp + 1) % 4, (jp + 2) % 4

        def drain_idx():
            pltpu.make_async_copy(ei_hbm.at[0], islot_v.at[sl_nxt, 0],
                                  isems[1 - p]).wait()
            pltpu.make_async_copy(ei_hbm.at[0], islot_v.at[sl_nxt, 1],
                                  isems[1 - p]).wait()

        def drain_scat():
            pltpu.make_async_copy(rows[1 - p], acc_s.at[islot_v.at[0, 1]],
                                  ssems[1 - p]).wait()

        def gather_nxt():
            pltpu.async_copy(y_hbm.at[islot_v.at[sl_nxt, 0]], rows[1 - p],
                             gsems[1 - p])

        def pre_idx():
            r = ibase + 2 * (j + 2)
            pltpu.async_copy(ei_hbm.at[r], islot_v.at[sl_pre, 0], isems[p])
            pltpu.async_copy(ei_hbm.at[r + 1], islot_v.at[sl_pre, 1],
                             isems[p])

        cond(j + 1 < CPT, drain_idx)
        cond(j >= 1, drain_scat)
        cond(j + 1 < CPT, gather_nxt)
        pltpu.make_async_copy(y_hbm.at[islot_v.at[sl_cur, 0]], rows[p],
                              gsems[p]).wait()
        pltpu.async_copy(rows[p], acc_s.at[islot_v.at[sl_cur, 1]], ssems[p],
                         add=True)
        cond(j + 2 < CPT, pre_idx)

    def body(k, _):
        for i in range(4):
            phase(4 * k + i, i)
        return 0

    lax.fori_loop(0, CPT // 4, body, 0)
    for j in range(4 * (CPT // 4), CPT):
        phase(j, j % 4)
    pltpu.make_async_copy(rows1_v, acc_s.at[islot_v.at[0, 1]], ssem1).wait()
    plsc.subcore_barrier()
    _copy_out(acc_s, zbuf_v, out_hbm, base, c * NP + base)


def _scales(d_ref):
    dout = d_ref[0, 0:N, 0:1]
    din = d_ref[0, 0:N, 64:65]
    s_out = lax.rsqrt(jnp.maximum(dout, 1.0))
    s_in = lax.rsqrt(jnp.maximum(din, 1.0))
    return s_out, s_in


def _tc_pre_body(x_ref, d_ref, o_ref):
    s_out, _ = _scales(d_ref)
    o_ref[0, 0:N, :] = x_ref[0] * s_out


_tc_pre = pl.pallas_call(
    _tc_pre_body,
    grid=(2,),
    in_specs=[
        pl.BlockSpec((1, N, D), lambda g: (g, 0, 0)),
        pl.BlockSpec((1, NP, D), lambda g: (g, 0, 0)),
    ],
    out_specs=pl.BlockSpec((1, NP, D), lambda g: (g, 0, 0)),
    out_shape=jax.ShapeDtypeStruct((2, NP, D), jnp.float32),
)


def _tc_mid_body(a_ref, d_ref, b_ref, w_ref, o_ref):
    s_out, s_in = _scales(d_ref)
    y = jnp.dot(a_ref[0, 0:N, :] * s_in, w_ref[...],
                preferred_element_type=jnp.float32)
    h = jnp.maximum(y + b_ref[0:1, :], 0.0)
    o_ref[0, 0:N, :] = h * s_out


_tc_mid = pl.pallas_call(
    _tc_mid_body,
    grid=(2,),
    in_specs=[
        pl.BlockSpec((1, NP, D), lambda g: (g, 0, 0)),
        pl.BlockSpec((1, NP, D), lambda g: (g, 0, 0)),
        pl.BlockSpec((8, D), lambda g: (0, 0)),
        pl.BlockSpec((D, D), lambda g: (0, 0)),
    ],
    out_specs=pl.BlockSpec((1, NP, D), lambda g: (g, 0, 0)),
    out_shape=jax.ShapeDtypeStruct((2, NP, D), jnp.float32),
)


def _tc_post_body(a_ref, d_ref, b_ref, w_ref, o_ref):
    _, s_in = _scales(d_ref)
    f = jnp.dot(a_ref[0, 0:N, :] * s_in, w_ref[...],
                preferred_element_type=jnp.float32) + b_ref[0:1, :]
    mu = jnp.mean(f, axis=0, keepdims=True)
    d = f - mu
    var = jnp.sum(d * d, axis=0, keepdims=True) * (1.0 / (N - 1))
    o_ref[0] = d / jnp.sqrt(var)


_tc_post = pl.pallas_call(
    _tc_post_body,
    grid=(2,),
    in_specs=[
        pl.BlockSpec((1, NP, D), lambda g: (g, 0, 0)),
        pl.BlockSpec((1, NP, D), lambda g: (g, 0, 0)),
        pl.BlockSpec((8, D), lambda g: (0, 0)),
        pl.BlockSpec((D, D), lambda g: (0, 0)),
    ],
    out_specs=pl.BlockSpec((1, N, D), lambda g: (g, 0, 0)),
    out_shape=jax.ShapeDtypeStruct((2, N, D), jnp.float32),
)


def kernel(feat1, feat2, edge_index1, edge_index2, W0, b0, W1, b1):
    npad = NE_PAD - NE

    def padv(v, val):
        return jnp.concatenate([v, jnp.full((npad,), val, jnp.int32)])

    src1, dst1 = edge_index1[0], edge_index1[1]
    src2, dst2 = edge_index2[0], edge_index2[1]

    def pack(sa, da):
        return jnp.stack([sa.reshape(NSUB * CPT, CH),
                          da.reshape(NSUB * CPT, CH)],
                         axis=1).reshape(2 * NSUB * CPT, CH)

    e_deg = jnp.concatenate([
        pack(padv(src1, DUMMY), padv(dst1, DUMMY)),
        pack(padv(src2, DUMMY), padv(dst2, DUMMY)),
    ])
    ei_gs = jnp.concatenate([
        pack(padv(src1, 0), padv(dst1, DUMMY)),
        pack(padv(src2, 0) + NP, padv(dst2, DUMMY)),
    ])

    col = jnp.arange(D, dtype=jnp.int32)
    onesl = jnp.broadcast_to((col < 64).astype(jnp.float32), (CH, D))
    onesr = jnp.broadcast_to((col >= 64).astype(jnp.float32), (CH, D))
    zerosd = jnp.zeros((ZROWS, D), jnp.float32)

    degs = _deg_kernel(e_deg, onesl, onesr, zerosd).reshape(2, NP, D)
    b0t = jnp.tile(b0[None, :], (8, 1))
    b1t = jnp.tile(b1[None, :], (8, 1))

    xs = jnp.stack([feat1, feat2])
    ys = _tc_pre(xs, degs)
    agg1 = _edge_kernel(ys.reshape(2 * NP, D), ei_gs,
                        zerosd).reshape(2, NP, D)
    ys2 = _tc_mid(agg1, degs, b0t, W0)
    agg2 = _edge_kernel(ys2.reshape(2 * NP, D), ei_gs,
                        zerosd).reshape(2, NP, D)
    z = _tc_post(agg2, degs, b1t, W1)
    return z[0], z[1]


# speedup vs baseline: 4.8973x; 1.0032x over previous
import functools

import jax
import jax.numpy as jnp
from jax import lax
from jax.experimental import pallas as pl
from jax.experimental.pallas import tpu as pltpu
from jax.experimental.pallas import tpu_sc as plsc

N = 10000
D = 128
NE = 320000
NP = 10240
DUMMY = NP - 1
NCORES = 2
NSUB = 16
CH = 128
CPT = 158
NE_PAD = NSUB * CPT * CH
RPT = NP // NSUB
ZROWS = 32

_MESH = plsc.VectorSubcoreMesh(
    core_axis_name="c", subcore_axis_name="s", num_cores=NCORES,
    num_subcores=NSUB)


def _zero_acc(zbuf_v, acc_s, base):
    for k in range(RPT // ZROWS):
        pltpu.sync_copy(zbuf_v, acc_s.at[pl.ds(base + k * ZROWS, ZROWS)])


def _copy_out(acc_s, zbuf_v, out_hbm, base, obase):
    for k in range(RPT // ZROWS):
        rows = pl.ds(base + k * ZROWS, ZROWS)
        out = pl.ds(obase + k * ZROWS, ZROWS)
        pltpu.sync_copy(acc_s.at[rows], zbuf_v)
        pltpu.sync_copy(zbuf_v, out_hbm.at[out])


@functools.partial(
    pl.kernel,
    out_type=jax.ShapeDtypeStruct((2 * NP, D), jnp.float32),
    mesh=_MESH,
    scratch_types=[
        pltpu.VMEM((CH, D), jnp.float32),
        pltpu.VMEM((CH, D), jnp.float32),
        pltpu.VMEM((ZROWS, D), jnp.float32),
        pltpu.VMEM((4, 2, CH), jnp.int32),
        pltpu.VMEM_SHARED((NP, D), jnp.float32),
        pltpu.SemaphoreType.DMA,
        pltpu.SemaphoreType.DMA,
        pltpu.SemaphoreType.DMA,
        pltpu.SemaphoreType.DMA,
    ],
)
def _deg_kernel(edges_hbm, onesl_hbm, onesr_hbm, zeros_hbm, out_hbm, onesl_v,
                onesr_v, zbuf_v, islot_v, acc_s, isem0, isem1, asem0, asem1):
    c = lax.axis_index("c")
    s = lax.axis_index("s")
    pltpu.sync_copy(onesl_hbm, onesl_v)
    pltpu.sync_copy(onesr_hbm, onesr_v)
    pltpu.sync_copy(zeros_hbm, zbuf_v)
    ibase = (c * NSUB + s) * 2 * CPT
    pltpu.sync_copy(edges_hbm.at[ibase], islot_v.at[0, 0])
    pltpu.sync_copy(edges_hbm.at[ibase + 1], islot_v.at[0, 1])
    pltpu.async_copy(edges_hbm.at[ibase + 2], islot_v.at[1, 0], isem1)
    pltpu.async_copy(edges_hbm.at[ibase + 3], islot_v.at[1, 1], isem1)

    base = s * RPT
    _zero_acc(zbuf_v, acc_s, base)
    plsc.subcore_barrier()

    isems = (isem0, isem1)
    asems = (asem0, asem1)

    def cond(flag, fn):
        if isinstance(flag, bool):
            if flag:
                fn()
        else:
            pl.when(flag)(fn)

    def phase(j, jp):
        p = jp % 2
        sl_cur, sl_pre = jp, (jp + 2) % 4

        def drain_adds():
            pltpu.make_async_copy(onesl_v, acc_s.at[islot_v.at[0, 1]],
                                  asems[p]).wait()
            pltpu.make_async_copy(onesr_v, acc_s.at[islot_v.at[0, 1]],
                                  asems[p]).wait()

        def drain_idx():
            pltpu.make_async_copy(edges_hbm.at[0], islot_v.at[sl_cur, 0],
                                  isems[p]).wait()
            pltpu.make_async_copy(edges_hbm.at[0], islot_v.at[sl_cur, 1],
                                  isems[p]).wait()

        def pre_idx():
            r = ibase + 2 * (j + 2)
            pltpu.async_copy(edges_hbm.at[r], islot_v.at[sl_pre, 0],
                             isems[p])
            pltpu.async_copy(edges_hbm.at[r + 1], islot_v.at[sl_pre, 1],
                             isems[p])

        cond(j >= 2, drain_adds)
        cond(j >= 1, drain_idx)
        pltpu.async_copy(onesl_v, acc_s.at[islot_v.at[sl_cur, 0]], asems[p],
                         add=True)
        pltpu.async_copy(onesr_v, acc_s.at[islot_v.at[sl_cur, 1]], asems[p],
                         add=True)
        cond(j + 2 < CPT, pre_idx)

    def body(k, _):
        for i in range(4):
            phase(4 * k + i, i)
        return 0

    lax.fori_loop(0, CPT // 4, body, 0)
    for j in range(4 * (CPT // 4), CPT):
        phase(j, j % 4)
    for sem in (asem0, asem1):
        pltpu.make_async_copy(onesl_v, acc_s.at[islot_v.at[0, 1]], sem).wait()
        pltpu.make_async_copy(onesr_v, acc_s.at[islot_v.at[0, 1]], sem).wait()
    plsc.subcore_barrier()
    _copy_out(acc_s, zbuf_v, out_hbm, base, c * NP + base)


@functools.partial(
    pl.kernel,
    out_type=jax.ShapeDtypeStruct((2 * NP, D), jnp.float32),
    mesh=_MESH,
    scratch_types=[
        pltpu.VMEM((4, 2, CH), jnp.int32),
        pltpu.VMEM((CH, D), jnp.float32),
        pltpu.VMEM((CH, D), jnp.float32),
        pltpu.VMEM((ZROWS, D), jnp.float32),
        pltpu.VMEM_SHARED((NP, D), jnp.float32),
        pltpu.SemaphoreType.DMA,
        pltpu.SemaphoreType.DMA,
        pltpu.SemaphoreType.DMA,
        pltpu.SemaphoreType.DMA,
        pltpu.SemaphoreType.DMA,
        pltpu.SemaphoreType.DMA,
    ],
)
def _edge_kernel(y_hbm, ei_hbm, zeros_hbm, out_hbm, islot_v, rows0_v, rows1_v,
                 zbuf_v, acc_s, gsem0, gsem1, isem0, isem1, ssem0, ssem1):
    c = lax.axis_index("c")
    s = lax.axis_index("s")
    pltpu.sync_copy(zeros_hbm, zbuf_v)
    ibase = (c * NSUB + s) * 2 * CPT
    pltpu.sync_copy(ei_hbm.at[ibase], islot_v.at[0, 0])
    pltpu.sync_copy(ei_hbm.at[ibase + 1], islot_v.at[0, 1])
    pltpu.async_copy(y_hbm.at[islot_v.at[0, 0]], rows0_v, gsem0)
    pltpu.async_copy(ei_hbm.at[ibase + 2], islot_v.at[1, 0], isem1)
    pltpu.async_copy(ei_hbm.at[ibase + 3], islot_v.at[1, 1], isem1)

    base = s * RPT
    _zero_acc(zbuf_v, acc_s, base)
    plsc.subcore_barrier()

    rows = (rows0_v, rows1_v)
    gsems = (gsem0, gsem1)
    isems = (isem0, isem1)
    ssems = (ssem0, ssem1)

    def cond(flag, fn):
        if isinstance(flag, bool):
            if flag:
                fn()
        else:
            pl.when(flag)(fn)

    def phase(j, jp):
        p = jp % 2
        sl_cur, sl_nxt, sl_pre = jp, (jp + 1) % 4, (jp + 2) % 4

        def drain_idx():
            pltpu.make_async_copy(ei_hbm.at[0], islot_v.at[sl_nxt, 0],
                                  isems[1 - p]).wait()
            pltpu.make_async_copy(ei_hbm.at[0], islot_v.at[sl_nxt, 1],
                                  isems[1 - p]).wait()

        def drain_scat():
            pltpu.make_async_copy(rows[1 - p], acc_s.at[islot_v.at[0, 1]],
                                  ssems[1 - p]).wait()

        def gather_nxt():
            pltpu.async_copy(y_hbm.at[islot_v.at[sl_nxt, 0]], rows[1 - p],
                             gsems[1 - p])

        def pre_idx():
            r = ibase + 2 * (j + 2)
            pltpu.async_copy(ei_hbm.at[r], islot_v.at[sl_pre, 0], isems[p])
            pltpu.async_copy(ei_hbm.at[r + 1], islot_v.at[sl_pre, 1],
                             isems[p])

        cond(j + 1 < CPT, drain_idx)
        cond(j >= 1, drain_scat)
        cond(j + 1 < CPT, gather_nxt)
        pltpu.make_async_copy(y_hbm.at[islot_v.at[sl_cur, 0]], rows[p],
                              gsems[p]).wait()
        pltpu.async_copy(rows[p], acc_s.at[islot_v.at[sl_cur, 1]], ssems[p],
                         add=True)
        cond(j + 2 < CPT, pre_idx)

    def body(k, _):
        for i in range(4):
            phase(4 * k + i, i)
        return 0

    lax.fori_loop(0, CPT // 4, body, 0)
    for j in range(4 * (CPT // 4), CPT):
        phase(j, j % 4)
    pltpu.make_async_copy(rows1_v, acc_s.at[islot_v.at[0, 1]], ssem1).wait()
    plsc.subcore_barrier()
    _copy_out(acc_s, zbuf_v, out_hbm, base, c * NP + base)


def _scales(d_ref):
    dout = d_ref[0, 0:N, 0:1]
    din = d_ref[0, 0:N, 64:65]
    s_out = lax.rsqrt(jnp.maximum(dout, 1.0))
    s_in = lax.rsqrt(jnp.maximum(din, 1.0))
    return s_out, s_in


def _tc_pre_body(x_ref, d_ref, o_ref):
    s_out, _ = _scales(d_ref)
    o_ref[0, 0:N, :] = x_ref[0] * s_out


_tc_pre = pl.pallas_call(
    _tc_pre_body,
    grid=(2,),
    in_specs=[
        pl.BlockSpec((1, N, D), lambda g: (g, 0, 0)),
        pl.BlockSpec((1, NP, D), lambda g: (g, 0, 0)),
    ],
    out_specs=pl.BlockSpec((1, NP, D), lambda g: (g, 0, 0)),
    out_shape=jax.ShapeDtypeStruct((2, NP, D), jnp.float32),
)


def _tc_mid_body(a_ref, d_ref, b_ref, w_ref, o_ref):
    s_out, s_in = _scales(d_ref)
    y = jnp.dot(a_ref[0, 0:N, :] * s_in, w_ref[...],
                preferred_element_type=jnp.float32)
    h = jnp.maximum(y + b_ref[0:1, :], 0.0)
    o_ref[0, 0:N, :] = h * s_out


_tc_mid = pl.pallas_call(
    _tc_mid_body,
    grid=(2,),
    in_specs=[
        pl.BlockSpec((1, NP, D), lambda g: (g, 0, 0)),
        pl.BlockSpec((1, NP, D), lambda g: (g, 0, 0)),
        pl.BlockSpec((8, D), lambda g: (0, 0)),
        pl.BlockSpec((D, D), lambda g: (0, 0)),
    ],
    out_specs=pl.BlockSpec((1, NP, D), lambda g: (g, 0, 0)),
    out_shape=jax.ShapeDtypeStruct((2, NP, D), jnp.float32),
)


def _tc_post_body(a_ref, d_ref, b_ref, w_ref, o_ref):
    _, s_in = _scales(d_ref)
    f = jnp.dot(a_ref[0, 0:N, :] * s_in, w_ref[...],
                preferred_element_type=jnp.float32) + b_ref[0:1, :]
    mu = jnp.mean(f, axis=0, keepdims=True)
    d = f - mu
    var = jnp.sum(d * d, axis=0, keepdims=True) * (1.0 / (N - 1))
    o_ref[0] = d / jnp.sqrt(var)


_tc_post = pl.pallas_call(
    _tc_post_body,
    grid=(2,),
    in_specs=[
        pl.BlockSpec((1, NP, D), lambda g: (g, 0, 0)),
        pl.BlockSpec((1, NP, D), lambda g: (g, 0, 0)),
        pl.BlockSpec((8, D), lambda g: (0, 0)),
        pl.BlockSpec((D, D), lambda g: (0, 0)),
    ],
    out_specs=pl.BlockSpec((1, N, D), lambda g: (g, 0, 0)),
    out_shape=jax.ShapeDtypeStruct((2, N, D), jnp.float32),
)


def kernel(feat1, feat2, edge_index1, edge_index2, W0, b0, W1, b1):
    npad = NE_PAD - NE

    def padv(v, val):
        return jnp.concatenate([v, jnp.full((npad,), val, jnp.int32)])

    src1, dst1 = edge_index1[0], edge_index1[1]
    src2, dst2 = edge_index2[0], edge_index2[1]

    def pack(sa, da):
        return jnp.stack([sa.reshape(NSUB * CPT, CH),
                          da.reshape(NSUB * CPT, CH)],
                         axis=1).reshape(2 * NSUB * CPT, CH)

    e_deg = jnp.concatenate([
        pack(padv(src1, DUMMY), padv(dst1, DUMMY)),
        pack(padv(src2, DUMMY), padv(dst2, DUMMY)),
    ])
    ei_gs = jnp.concatenate([
        pack(padv(src1, 0), padv(dst1, DUMMY)),
        pack(padv(src2, 0) + NP, padv(dst2, DUMMY)),
    ])

    col = jnp.arange(D, dtype=jnp.int32)
    onesl = jnp.broadcast_to((col < 64).astype(jnp.float32), (CH, D))
    onesr = jnp.broadcast_to((col >= 64).astype(jnp.float32), (CH, D))
    zerosd = jnp.zeros((ZROWS, D), jnp.float32)

    degs = _deg_kernel(e_deg, onesl, onesr, zerosd).reshape(2, NP, D)
    b0t = jnp.tile(b0[None, :], (8, 1))
    b1t = jnp.tile(b1[None, :], (8, 1))

    xs = jnp.stack([feat1, feat2])
    ys = _tc_pre(xs, degs)
    agg1 = _edge_kernel(ys.reshape(2 * NP, D), ei_gs,
                        zerosd).reshape(2, NP, D)
    ys2 = _tc_mid(agg1, degs, b0t, W0)
    agg2 = _edge_kernel(ys2.reshape(2 * NP, D), ei_gs,
                        zerosd).reshape(2, NP, D)
    z = _tc_post(agg2, degs, b1t, W1)
    return z[0], z[1]


# speedup vs baseline: 4.9229x; 1.0052x over previous
import functools

import jax
import jax.numpy as jnp
from jax import lax
from jax.experimental import pallas as pl
from jax.experimental.pallas import tpu as pltpu
from jax.experimental.pallas import tpu_sc as plsc

N = 10000
D = 128
NE = 320000
NP = 10240
DUMMY = NP - 1
NCORES = 2
NSUB = 16
CH = 128
CPT = 158
NE_PAD = NSUB * CPT * CH
RPT = NP // NSUB
ZROWS = 32

_MESH = plsc.VectorSubcoreMesh(
    core_axis_name="c", subcore_axis_name="s", num_cores=NCORES,
    num_subcores=NSUB)


def _zero_acc(zbuf_v, acc_s, base):
    for k in range(RPT // ZROWS):
        pltpu.sync_copy(zbuf_v, acc_s.at[pl.ds(base + k * ZROWS, ZROWS)])


def _copy_out(acc_s, zbuf_v, out_hbm, base, obase):
    for k in range(RPT // ZROWS):
        rows = pl.ds(base + k * ZROWS, ZROWS)
        out = pl.ds(obase + k * ZROWS, ZROWS)
        pltpu.sync_copy(acc_s.at[rows], zbuf_v)
        pltpu.sync_copy(zbuf_v, out_hbm.at[out])


@functools.partial(
    pl.kernel,
    out_type=jax.ShapeDtypeStruct((2 * NP, D), jnp.float32),
    mesh=_MESH,
    scratch_types=[
        pltpu.VMEM((CH, D), jnp.float32),
        pltpu.VMEM((CH, D), jnp.float32),
        pltpu.VMEM((ZROWS, D), jnp.float32),
        pltpu.VMEM((4, 2, CH), jnp.int32),
        pltpu.VMEM_SHARED((NP, D), jnp.float32),
        pltpu.SemaphoreType.DMA,
        pltpu.SemaphoreType.DMA,
        pltpu.SemaphoreType.DMA,
        pltpu.SemaphoreType.DMA,
    ],
)
def _deg_kernel(edges_hbm, onesl_hbm, onesr_hbm, zeros_hbm, out_hbm, onesl_v,
                onesr_v, zbuf_v, islot_v, acc_s, isem0, isem1, asem0, asem1):
    c = lax.axis_index("c")
    s = lax.axis_index("s")
    pltpu.sync_copy(onesl_hbm, onesl_v)
    pltpu.sync_copy(onesr_hbm, onesr_v)
    pltpu.sync_copy(zeros_hbm, zbuf_v)
    ibase = (c * NSUB + s) * 2 * CPT
    pltpu.sync_copy(edges_hbm.at[ibase], islot_v.at[0, 0])
    pltpu.sync_copy(edges_hbm.at[ibase + 1], islot_v.at[0, 1])
    pltpu.async_copy(edges_hbm.at[ibase + 2], islot_v.at[1, 0], isem1)
    pltpu.async_copy(edges_hbm.at[ibase + 3], islot_v.at[1, 1], isem1)

    base = s * RPT
    _zero_acc(zbuf_v, acc_s, base)
    plsc.subcore_barrier()

    isems = (isem0, isem1)
    asems = (asem0, asem1)

    def cond(flag, fn):
        if isinstance(flag, bool):
            if flag:
                fn()
        else:
            pl.when(flag)(fn)

    def phase(j, jp):
        p = jp % 2
        sl_cur, sl_pre = jp, (jp + 2) % 4

        def drain_adds():
            pltpu.make_async_copy(onesl_v, acc_s.at[islot_v.at[0, 1]],
                                  asems[p]).wait()
            pltpu.make_async_copy(onesr_v, acc_s.at[islot_v.at[0, 1]],
                                  asems[p]).wait()

        def drain_idx():
            pltpu.make_async_copy(edges_hbm.at[0], islot_v.at[sl_cur, 0],
                                  isems[p]).wait()
            pltpu.make_async_copy(edges_hbm.at[0], islot_v.at[sl_cur, 1],
                                  isems[p]).wait()

        def pre_idx():
            r = ibase + 2 * (j + 2)
            pltpu.async_copy(edges_hbm.at[r], islot_v.at[sl_pre, 0],
                             isems[p])
            pltpu.async_copy(edges_hbm.at[r + 1], islot_v.at[sl_pre, 1],
                             isems[p])

        cond(j >= 2, drain_adds)
        cond(j >= 1, drain_idx)
        pltpu.async_copy(onesl_v, acc_s.at[islot_v.at[sl_cur, 0]], asems[p],
                         add=True)
        pltpu.async_copy(onesr_v, acc_s.at[islot_v.at[sl_cur, 1]], asems[p],
                         add=True)
        cond(j + 2 < CPT, pre_idx)

    def body(k, _):
        for i in range(4):
            phase(4 * k + i, i)
        return 0

    lax.fori_loop(0, CPT // 4, body, 0)
    for j in range(4 * (CPT // 4), CPT):
        phase(j, j % 4)
    for sem in (asem0, asem1):
        pltpu.make_async_copy(onesl_v, acc_s.at[islot_v.at[0, 1]], sem).wait()
        pltpu.make_async_copy(onesr_v, acc_s.at[islot_v.at[0, 1]], sem).wait()
    plsc.subcore_barrier()
    _copy_out(acc_s, zbuf_v, out_hbm, base, c * NP + base)


@functools.partial(
    pl.kernel,
    out_type=jax.ShapeDtypeStruct((2 * NP, D), jnp.float32),
    mesh=_MESH,
    scratch_types=[
        pltpu.VMEM((4, 2, CH), jnp.int32),
        pltpu.VMEM((CH, D), jnp.float32),
        pltpu.VMEM((CH, D), jnp.float32),
        pltpu.VMEM((ZROWS, D), jnp.float32),
        pltpu.VMEM_SHARED((NP, D), jnp.float32),
        pltpu.SemaphoreType.DMA,
        pltpu.SemaphoreType.DMA,
        pltpu.SemaphoreType.DMA,
        pltpu.SemaphoreType.DMA,
        pltpu.SemaphoreType.DMA,
        pltpu.SemaphoreType.DMA,
    ],
)
def _edge_kernel(y_hbm, ei_hbm, zeros_hbm, out_hbm, islot_v, rows0_v, rows1_v,
                 zbuf_v, acc_s, gsem0, gsem1, isem0, isem1, ssem0, ssem1):
    c = lax.axis_index("c")
    s = lax.axis_index("s")
    pltpu.sync_copy(zeros_hbm, zbuf_v)
    ibase = (c * NSUB + s) * 2 * CPT
    pltpu.sync_copy(ei_hbm.at[ibase], islot_v.at[0, 0])
    pltpu.sync_copy(ei_hbm.at[ibase + 1], islot_v.at[0, 1])
    pltpu.async_copy(y_hbm.at[islot_v.at[0, 0]], rows0_v, gsem0)
    pltpu.async_copy(ei_hbm.at[ibase + 2], islot_v.at[1, 0], isem1)
    pltpu.async_copy(ei_hbm.at[ibase + 3], islot_v.at[1, 1], isem1)

    base = s * RPT
    _zero_acc(zbuf_v, acc_s, base)
    plsc.subcore_barrier()

    rows = (rows0_v, rows1_v)
    gsems = (gsem0, gsem1)
    isems = (isem0, isem1)
    ssems = (ssem0, ssem1)

    def cond(flag, fn):
        if isinstance(flag, bool):
            if flag:
                fn()
        else:
            pl.when(flag)(fn)

    def phase(j, jp):
        p = jp % 2
        sl_cur, sl_nxt, sl_pre = jp, (jp + 1) % 4, (jp + 2) % 4

        def drain_idx():
            pltpu.make_async_copy(ei_hbm.at[0], islot_v.at[sl_nxt, 0],
                                  isems[1 - p]).wait()
            pltpu.make_async_copy(ei_hbm.at[0], islot_v.at[sl_nxt, 1],
                                  isems[1 - p]).wait()

        def drain_scat():
            pltpu.make_async_copy(rows[1 - p], acc_s.at[islot_v.at[0, 1]],
                                  ssems[1 - p]).wait()

        def gather_nxt():
            pltpu.async_copy(y_hbm.at[islot_v.at[sl_nxt, 0]], rows[1 - p],
                             gsems[1 - p])

        def pre_idx():
            r = ibase + 2 * (j + 2)
            pltpu.async_copy(ei_hbm.at[r], islot_v.at[sl_pre, 0], isems[p])
            pltpu.async_copy(ei_hbm.at[r + 1], islot_v.at[sl_pre, 1],
                             isems[p])

        cond(j + 2 < CPT, pre_idx)
        cond(j + 1 < CPT, drain_idx)
        cond(j >= 1, drain_scat)
        cond(j + 1 < CPT, gather_nxt)
        pltpu.make_async_copy(y_hbm.at[islot_v.at[sl_cur, 0]], rows[p],
                              gsems[p]).wait()
        pltpu.async_copy(rows[p], acc_s.at[islot_v.at[sl_cur, 1]], ssems[p],
                         add=True)

    def body(k, _):
        for i in range(4):
            phase(4 * k + i, i)
        return 0

    lax.fori_loop(0, CPT // 4, body, 0)
    for j in range(4 * (CPT // 4), CPT):
        phase(j, j % 4)
    pltpu.make_async_copy(rows1_v, acc_s.at[islot_v.at[0, 1]], ssem1).wait()
    plsc.subcore_barrier()
    _copy_out(acc_s, zbuf_v, out_hbm, base, c * NP + base)


def _scales(d_ref):
    dout = d_ref[0, 0:N, 0:1]
    din = d_ref[0, 0:N, 64:65]
    s_out = lax.rsqrt(jnp.maximum(dout, 1.0))
    s_in = lax.rsqrt(jnp.maximum(din, 1.0))
    return s_out, s_in


def _tc_pre_body(x_ref, d_ref, o_ref):
    s_out, _ = _scales(d_ref)
    o_ref[0, 0:N, :] = x_ref[0] * s_out


_tc_pre = pl.pallas_call(
    _tc_pre_body,
    grid=(2,),
    in_specs=[
        pl.BlockSpec((1, N, D), lambda g: (g, 0, 0)),
        pl.BlockSpec((1, NP, D), lambda g: (g, 0, 0)),
    ],
    out_specs=pl.BlockSpec((1, NP, D), lambda g: (g, 0, 0)),
    out_shape=jax.ShapeDtypeStruct((2, NP, D), jnp.float32),
)


def _tc_mid_body(a_ref, d_ref, b_ref, w_ref, o_ref):
    s_out, s_in = _scales(d_ref)
    y = jnp.dot(a_ref[0, 0:N, :] * s_in, w_ref[...],
                preferred_element_type=jnp.float32)
    h = jnp.maximum(y + b_ref[0:1, :], 0.0)
    o_ref[0, 0:N, :] = h * s_out


_tc_mid = pl.pallas_call(
    _tc_mid_body,
    grid=(2,),
    in_specs=[
        pl.BlockSpec((1, NP, D), lambda g: (g, 0, 0)),
        pl.BlockSpec((1, NP, D), lambda g: (g, 0, 0)),
        pl.BlockSpec((8, D), lambda g: (0, 0)),
        pl.BlockSpec((D, D), lambda g: (0, 0)),
    ],
    out_specs=pl.BlockSpec((1, NP, D), lambda g: (g, 0, 0)),
    out_shape=jax.ShapeDtypeStruct((2, NP, D), jnp.float32),
)


def _tc_post_body(a_ref, d_ref, b_ref, w_ref, o_ref):
    _, s_in = _scales(d_ref)
    f = jnp.dot(a_ref[0, 0:N, :] * s_in, w_ref[...],
                preferred_element_type=jnp.float32) + b_ref[0:1, :]
    mu = jnp.mean(f, axis=0, keepdims=True)
    d = f - mu
    var = jnp.sum(d * d, axis=0, keepdims=True) * (1.0 / (N - 1))
    o_ref[0] = d / jnp.sqrt(var)


_tc_post = pl.pallas_call(
    _tc_post_body,
    grid=(2,),
    in_specs=[
        pl.BlockSpec((1, NP, D), lambda g: (g, 0, 0)),
        pl.BlockSpec((1, NP, D), lambda g: (g, 0, 0)),
        pl.BlockSpec((8, D), lambda g: (0, 0)),
        pl.BlockSpec((D, D), lambda g: (0, 0)),
    ],
    out_specs=pl.BlockSpec((1, N, D), lambda g: (g, 0, 0)),
    out_shape=jax.ShapeDtypeStruct((2, N, D), jnp.float32),
)


def kernel(feat1, feat2, edge_index1, edge_index2, W0, b0, W1, b1):
    npad = NE_PAD - NE

    def padv(v, val):
        return jnp.concatenate([v, jnp.full((npad,), val, jnp.int32)])

    src1, dst1 = edge_index1[0], edge_index1[1]
    src2, dst2 = edge_index2[0], edge_index2[1]

    def pack(sa, da):
        return jnp.stack([sa.reshape(NSUB * CPT, CH),
                          da.reshape(NSUB * CPT, CH)],
                         axis=1).reshape(2 * NSUB * CPT, CH)

    e_deg = jnp.concatenate([
        pack(padv(src1, DUMMY), padv(dst1, DUMMY)),
        pack(padv(src2, DUMMY), padv(dst2, DUMMY)),
    ])
    ei_gs = jnp.concatenate([
        pack(padv(src1, 0), padv(dst1, DUMMY)),
        pack(padv(src2, 0) + NP, padv(dst2, DUMMY)),
    ])

    col = jnp.arange(D, dtype=jnp.int32)
    onesl = jnp.broadcast_to((col < 64).astype(jnp.float32), (CH, D))
    onesr = jnp.broadcast_to((col >= 64).astype(jnp.float32), (CH, D))
    zerosd = jnp.zeros((ZROWS, D), jnp.float32)

    degs = _deg_kernel(e_deg, onesl, onesr, zerosd).reshape(2, NP, D)
    b0t = jnp.tile(b0[None, :], (8, 1))
    b1t = jnp.tile(b1[None, :], (8, 1))

    xs = jnp.stack([feat1, feat2])
    ys = _tc_pre(xs, degs)
    agg1 = _edge_kernel(ys.reshape(2 * NP, D), ei_gs,
                        zerosd).reshape(2, NP, D)
    ys2 = _tc_mid(agg1, degs, b0t, W0)
    agg2 = _edge_kernel(ys2.reshape(2 * NP, D), ei_gs,
                        zerosd).reshape(2, NP, D)
    z = _tc_post(agg2, degs, b1t, W1)
    return z[0], z[1]
